# baseline (speedup 1.0000x reference)
import functools

import jax
import jax.numpy as jnp
from jax import lax
from jax.experimental import pallas as pl
from jax.experimental.pallas import tpu as pltpu
from jax.experimental.pallas import tpu_sc as plsc

N_NODES = 100000
DIM = 128
B_POS = 16384
B_NEG = 81920
EPS = 1e-7

E = B_POS + B_NEG
TOT = 2 * E
NW = 32
PER_W = E // NW
CH = 128
N_CH = PER_W // CH

LOG2E = 1.4426950408889634
LN2 = 0.6931471805599453
EPS2 = EPS / LN2

N_PAD = 100096
ROWS_TAB = N_PAD // 128


def _softplus2(x):
    return jnp.log2(1.0 + jnp.exp2(x * LOG2E))


def _tab_body(o_ref, sp_ref, lvl2_ref):
    x = o_ref[...]
    sp = LN2 * _softplus2(x)
    sp_ref[...] = sp
    lvl2_ref[...] = jnp.log2(_softplus2(sp) + EPS2)


_tab_kernel = pl.pallas_call(
    _tab_body,
    out_shape=[
        jax.ShapeDtypeStruct((ROWS_TAB, 128), jnp.float32),
        jax.ShapeDtypeStruct((ROWS_TAB, 128), jnp.float32),
    ],
)


@functools.cache
def _make_sc_inter(ebase, ecount):
    mesh = plsc.VectorSubcoreMesh(core_axis_name="c", subcore_axis_name="s")

    per_w = ecount // NW
    n_ch = per_w // CH
    NB = 3
    scratch_types = [
        pltpu.VMEM((per_w,), jnp.int32),
        pltpu.VMEM((per_w,), jnp.int32),
        pltpu.VMEM((per_w,), jnp.float32),
        pltpu.VMEM((per_w,), jnp.float32),
        pltpu.VMEM((per_w,), jnp.float32),
    ]
    scratch_types += [pltpu.VMEM((CH, DIM), jnp.float32)] * (2 * NB)
    scratch_types += [pltpu.SemaphoreType.DMA] * (3 * NB + 1)

    @functools.partial(
        pl.kernel,
        mesh=mesh,
        out_type=(
            jax.ShapeDtypeStruct((ecount, DIM), jnp.float32),
            jax.ShapeDtypeStruct((ecount,), jnp.float32),
        ),
        scratch_types=scratch_types,
    )
    def _sc_inter(embs_hbm, sp_hbm, lvl2_hbm, idx_hbm, d_out, lvl2_out,
                  idxu, idxv, spu, spv, lvl2v, *scr):
        PER_W = per_w
        N_CH = n_ch
        ubufs = scr[:NB]
        vbufs = scr[NB:2 * NB]
        sgu = scr[2 * NB:3 * NB]
        sgv = scr[3 * NB:4 * NB]
        sws = scr[4 * NB:5 * NB]
        so = scr[5 * NB]
        wid = lax.axis_index("s") * 2 + lax.axis_index("c")
        base = wid * PER_W
        pltpu.sync_copy(idx_hbm.at[pl.ds(ebase + base, PER_W)], idxu)
        pltpu.sync_copy(idx_hbm.at[pl.ds(E + ebase + base, PER_W)], idxv)

        def sbody(j, carry):
            sl = pl.ds(j * CH, CH)
            pltpu.async_copy(sp_hbm.at[idxu.at[sl]], spu.at[sl], so)
            pltpu.async_copy(sp_hbm.at[idxv.at[sl]], spv.at[sl], so)
            pltpu.async_copy(lvl2_hbm.at[idxv.at[sl]], lvl2v.at[sl], so)
            return carry

        lax.fori_loop(0, N_CH, sbody, 0)

        def start_g(c, p):
            sl = pl.ds(c * CH, CH)
            pltpu.async_copy(embs_hbm.at[idxu.at[sl]], ubufs[p], sgu[p])
            pltpu.async_copy(embs_hbm.at[idxv.at[sl]], vbufs[p], sgv[p])

        def wait_g(c, p):
            sl = pl.ds(c * CH, CH)
            pltpu.make_async_copy(embs_hbm.at[idxu.at[sl]], ubufs[p],
                                  sgu[p]).wait()
            pltpu.make_async_copy(embs_hbm.at[idxv.at[sl]], vbufs[p],
                                  sgv[p]).wait()

        def out_at(c):
            return d_out.at[pl.ds(base + c * CH, CH)]

        def start_w(c, p):
            pltpu.async_copy(ubufs[p], out_at(c), sws[p])

        def wait_w(c, p):
            pltpu.make_async_copy(ubufs[p], out_at(c), sws[p]).wait()

        start_g(0, 0)
        pltpu.make_async_copy(sp_hbm.at[pl.ds(0, PER_W)], spu, so).wait()
        pltpu.make_async_copy(sp_hbm.at[pl.ds(0, PER_W)], spv, so).wait()
        pltpu.make_async_copy(lvl2_hbm.at[pl.ds(0, PER_W)], lvl2v, so).wait()

        def cbody(co, carry):
            for kb in range(NB):
                c = co * NB + kb
                pn = (kb + 1) % NB

                @pl.when(c + 1 < N_CH)
                def _():
                    @pl.when(c >= 2)
                    def _():
                        wait_w(c - 2, pn)
                    start_g(c + 1, pn)

                wait_g(c, kb)
                ub = ubufs[kb]
                vb = vbufs[kb]

                def gbody(g, gcarry, ub=ub, vb=vb, c=c):
                    suv = spu[pl.ds(c * CH + g * 16, 16)]
                    svv = spv[pl.ds(c * CH + g * 16, 16)]
                    for j in range(16):
                        r = g * 16 + j
                        su = jnp.full((16,), suv[j], jnp.float32)
                        sv = jnp.full((16,), svv[j], jnp.float32)
                        for k in range(DIM // 16):
                            sl = pl.ds(k * 16, 16)
                            uu = ub[r, sl]
                            vv = vb[r, sl]
                            ub[r, sl] = jnp.minimum(uu + su, vv + sv) \
                                - jnp.maximum(uu, vv)
                    return gcarry

                lax.fori_loop(0, CH // 16, gbody, 0)
                start_w(c, kb)
            return carry

        lax.fori_loop(0, N_CH // NB, cbody, 0)
        for c in range(N_CH - NB, N_CH):
            wait_w(c, c % NB)
        pltpu.sync_copy(lvl2v, lvl2_out.at[pl.ds(base, PER_W)])

    return _sc_inter


ROWS_E = E // 128

BLK = 8192
NBLK = E // BLK
NPOS = B_POS // BLK
BRK = BLK // 128


def _math_body(npos, d_ref, lvl2_ref, prob_ref, pos_ref, neg_ref):
    i = pl.program_id(0)
    out_rows = []
    for r in range(BRK):
        dt = d_ref[r * 128:(r + 1) * 128, :].T
        t = jnp.exp2(dt * LOG2E)
        s = jnp.log2(jnp.log2(1.0 + t) + EPS2)
        out_rows.append(jnp.sum(s, axis=0, keepdims=True))
    sums = jnp.concatenate(out_rows, axis=0)
    logp = LN2 * (sums - DIM * lvl2_ref[...])
    prob = jax.nn.sigmoid(logp)
    prob_ref[...] = prob
    lp = jnp.sum(jnp.log(prob + EPS))
    ln = jnp.sum(jnp.log(1.0 - prob + EPS))

    @pl.when(i == 0)
    def _():
        pos_ref[0, 0] = 0.0
        neg_ref[0, 0] = 0.0

    is_pos = i < npos
    pos_ref[0, 0] += jnp.where(is_pos, lp, 0.0)
    neg_ref[0, 0] += jnp.where(is_pos, 0.0, ln)


@functools.cache
def _make_math(ecount, npos):
    nblk = ecount // BLK
    rows = ecount // 128
    return pl.pallas_call(
        functools.partial(_math_body, npos),
        grid=(nblk,),
        in_specs=[
            pl.BlockSpec((BLK, DIM), lambda i: (i, 0)),
            pl.BlockSpec((BRK, 128), lambda i: (i, 0)),
        ],
        out_specs=[
            pl.BlockSpec((BRK, 128), lambda i: (i, 0)),
            pl.BlockSpec(memory_space=pltpu.SMEM),
            pl.BlockSpec(memory_space=pltpu.SMEM),
        ],
        out_shape=[
            jax.ShapeDtypeStruct((rows, 128), jnp.float32),
            jax.ShapeDtypeStruct((1, 1), jnp.float32),
            jax.ShapeDtypeStruct((1, 1), jnp.float32),
        ],
    )


EH = E // 2


def kernel(embs, offset_embs, pos_u, pos_v, neg_u, neg_v):
    idx = jnp.concatenate([pos_u, neg_u, pos_v, neg_v]).astype(jnp.int32)
    off_pad = jnp.pad(offset_embs.reshape(N_NODES),
                      (0, N_PAD - N_NODES)).reshape(ROWS_TAB, 128)
    sp_t, lvl2_t = _tab_kernel(off_pad)
    sp_f = sp_t.reshape(N_PAD)
    lvl2_f = lvl2_t.reshape(N_PAD)
    d0, l0 = _make_sc_inter(0, EH)(embs, sp_f, lvl2_f, idx)
    d1, l1 = _make_sc_inter(EH, EH)(embs, sp_f, lvl2_f, idx)
    p0, ps0, ns0 = _make_math(EH, B_POS // BLK)(d0, l0.reshape(EH // 128, 128))
    p1, ps1, ns1 = _make_math(EH, 0)(d1, l1.reshape(EH // 128, 128))
    loss = -((ps0[0, 0] + ps1[0, 0]) / B_POS) \
        - ((ns0[0, 0] + ns1[0, 0]) / B_NEG)
    edge_prob = jnp.concatenate([p0.reshape(EH), p1.reshape(EH)])
    ground_truth = jnp.concatenate(
        [jnp.ones(B_POS, dtype=jnp.float32), jnp.zeros(B_NEG, dtype=jnp.float32)])
    return loss, edge_prob, ground_truth


# speedup vs baseline: 1.0462x; 1.0462x over previous
import functools

import jax
import jax.numpy as jnp
from jax import lax
from jax.experimental import pallas as pl
from jax.experimental.pallas import tpu as pltpu
from jax.experimental.pallas import tpu_sc as plsc

N_NODES = 100000
DIM = 128
B_POS = 16384
B_NEG = 81920
EPS = 1e-7

E = B_POS + B_NEG
TOT = 2 * E
NW = 32
PER_W = E // NW
CH = 128
N_CH = PER_W // CH

LOG2E = 1.4426950408889634
LN2 = 0.6931471805599453
EPS2 = EPS / LN2

N_PAD = 100096
ROWS_TAB = N_PAD // 128


def _softplus2(x):
    return jnp.log2(1.0 + jnp.exp2(x * LOG2E))


def _tab_body(o_ref, sp_ref, lvl2_ref):
    x = o_ref[...]
    sp = LN2 * _softplus2(x)
    sp_ref[...] = sp
    lvl2_ref[...] = jnp.log2(_softplus2(sp) + EPS2)


_tab_kernel = pl.pallas_call(
    _tab_body,
    out_shape=[
        jax.ShapeDtypeStruct((ROWS_TAB, 128), jnp.float32),
        jax.ShapeDtypeStruct((ROWS_TAB, 128), jnp.float32),
    ],
)


@functools.cache
def _make_sc_inter(ebase, ecount):
    mesh = plsc.VectorSubcoreMesh(core_axis_name="c", subcore_axis_name="s")

    per_w = ecount // NW
    n_ch = per_w // CH
    NB = 3
    scratch_types = [
        pltpu.VMEM((per_w,), jnp.int32),
        pltpu.VMEM((per_w,), jnp.int32),
        pltpu.VMEM((per_w,), jnp.float32),
        pltpu.VMEM((per_w,), jnp.float32),
        pltpu.VMEM((per_w,), jnp.float32),
    ]
    scratch_types += [pltpu.VMEM((CH, DIM), jnp.float32)] * (2 * NB)
    scratch_types += [pltpu.SemaphoreType.DMA] * (3 * NB + 1)

    @functools.partial(
        pl.kernel,
        mesh=mesh,
        out_type=(
            jax.ShapeDtypeStruct((ecount, DIM), jnp.float32),
            jax.ShapeDtypeStruct((ecount,), jnp.float32),
        ),
        scratch_types=scratch_types,
    )
    def _sc_inter(embs_hbm, sp_hbm, lvl2_hbm, idx_hbm, d_out, lvl2_out,
                  idxu, idxv, spu, spv, lvl2v, *scr):
        PER_W = per_w
        N_CH = n_ch
        ubufs = scr[:NB]
        vbufs = scr[NB:2 * NB]
        sgu = scr[2 * NB:3 * NB]
        sgv = scr[3 * NB:4 * NB]
        sws = scr[4 * NB:5 * NB]
        so = scr[5 * NB]
        wid = lax.axis_index("s") * 2 + lax.axis_index("c")
        base = wid * PER_W
        pltpu.sync_copy(idx_hbm.at[pl.ds(ebase + base, PER_W)], idxu)
        pltpu.sync_copy(idx_hbm.at[pl.ds(E + ebase + base, PER_W)], idxv)

        def start_g(c, p):
            sl = pl.ds(c * CH, CH)
            pltpu.async_copy(embs_hbm.at[idxu.at[sl]], ubufs[p], sgu[p])
            pltpu.async_copy(embs_hbm.at[idxv.at[sl]], vbufs[p], sgv[p])

        def wait_g(c, p):
            sl = pl.ds(c * CH, CH)
            pltpu.make_async_copy(embs_hbm.at[idxu.at[sl]], ubufs[p],
                                  sgu[p]).wait()
            pltpu.make_async_copy(embs_hbm.at[idxv.at[sl]], vbufs[p],
                                  sgv[p]).wait()

        def out_at(c):
            return d_out.at[pl.ds(base + c * CH, CH)]

        def start_w(c, p):
            pltpu.async_copy(ubufs[p], out_at(c), sws[p])

        def wait_w(c, p):
            pltpu.make_async_copy(ubufs[p], out_at(c), sws[p]).wait()

        start_g(0, 0)
        SCH = 1024
        for j in range(PER_W // SCH):
            sl = pl.ds(j * SCH, SCH)
            pltpu.async_copy(sp_hbm.at[idxu.at[sl]], spu.at[sl], so)
            pltpu.async_copy(sp_hbm.at[idxv.at[sl]], spv.at[sl], so)
            pltpu.async_copy(lvl2_hbm.at[idxv.at[sl]], lvl2v.at[sl], so)
        pltpu.make_async_copy(sp_hbm.at[pl.ds(0, PER_W)], spu, so).wait()
        pltpu.make_async_copy(sp_hbm.at[pl.ds(0, PER_W)], spv, so).wait()
        pltpu.make_async_copy(lvl2_hbm.at[pl.ds(0, PER_W)], lvl2v, so).wait()

        def cbody(co, carry):
            for kb in range(NB):
                c = co * NB + kb
                pn = (kb + 1) % NB

                @pl.when(c + 1 < N_CH)
                def _():
                    @pl.when(c >= 2)
                    def _():
                        wait_w(c - 2, pn)
                    start_g(c + 1, pn)

                wait_g(c, kb)
                ub = ubufs[kb]
                vb = vbufs[kb]

                def gbody(g, gcarry, ub=ub, vb=vb, c=c):
                    suv = spu[pl.ds(c * CH + g * 16, 16)]
                    svv = spv[pl.ds(c * CH + g * 16, 16)]
                    for j in range(16):
                        r = g * 16 + j
                        su = jnp.full((16,), suv[j], jnp.float32)
                        sv = jnp.full((16,), svv[j], jnp.float32)
                        for k in range(DIM // 16):
                            sl = pl.ds(k * 16, 16)
                            uu = ub[r, sl]
                            vv = vb[r, sl]
                            ub[r, sl] = jnp.minimum(uu + su, vv + sv) \
                                - jnp.maximum(uu, vv)
                    return gcarry

                lax.fori_loop(0, CH // 16, gbody, 0)
                start_w(c, kb)
            return carry

        lax.fori_loop(0, N_CH // NB, cbody, 0)
        for c in range(N_CH - NB, N_CH):
            wait_w(c, c % NB)
        pltpu.sync_copy(lvl2v, lvl2_out.at[pl.ds(base, PER_W)])

    return _sc_inter


ROWS_E = E // 128

BLK = 8192
NBLK = E // BLK
NPOS = B_POS // BLK
BRK = BLK // 128


def _math_body(npos, d_ref, lvl2_ref, prob_ref, pos_ref, neg_ref):
    i = pl.program_id(0)
    out_rows = []
    for r in range(BRK):
        dt = d_ref[r * 128:(r + 1) * 128, :].T
        t = jnp.exp2(dt * LOG2E)
        s = jnp.log2(jnp.log2(1.0 + t) + EPS2)
        out_rows.append(jnp.sum(s, axis=0, keepdims=True))
    sums = jnp.concatenate(out_rows, axis=0)
    logp = LN2 * (sums - DIM * lvl2_ref[...])
    prob = jax.nn.sigmoid(logp)
    prob_ref[...] = prob
    lp = jnp.sum(jnp.log(prob + EPS))
    ln = jnp.sum(jnp.log(1.0 - prob + EPS))

    @pl.when(i == 0)
    def _():
        pos_ref[0, 0] = 0.0
        neg_ref[0, 0] = 0.0

    is_pos = i < npos
    pos_ref[0, 0] += jnp.where(is_pos, lp, 0.0)
    neg_ref[0, 0] += jnp.where(is_pos, 0.0, ln)


@functools.cache
def _make_math(ecount, npos):
    nblk = ecount // BLK
    rows = ecount // 128
    return pl.pallas_call(
        functools.partial(_math_body, npos),
        grid=(nblk,),
        in_specs=[
            pl.BlockSpec((BLK, DIM), lambda i: (i, 0)),
            pl.BlockSpec((BRK, 128), lambda i: (i, 0)),
        ],
        out_specs=[
            pl.BlockSpec((BRK, 128), lambda i: (i, 0)),
            pl.BlockSpec(memory_space=pltpu.SMEM),
            pl.BlockSpec(memory_space=pltpu.SMEM),
        ],
        out_shape=[
            jax.ShapeDtypeStruct((rows, 128), jnp.float32),
            jax.ShapeDtypeStruct((1, 1), jnp.float32),
            jax.ShapeDtypeStruct((1, 1), jnp.float32),
        ],
    )


EH = E // 2


def kernel(embs, offset_embs, pos_u, pos_v, neg_u, neg_v):
    idx = jnp.concatenate([pos_u, neg_u, pos_v, neg_v]).astype(jnp.int32)
    off_pad = jnp.pad(offset_embs.reshape(N_NODES),
                      (0, N_PAD - N_NODES)).reshape(ROWS_TAB, 128)
    sp_t, lvl2_t = _tab_kernel(off_pad)
    sp_f = sp_t.reshape(N_PAD)
    lvl2_f = lvl2_t.reshape(N_PAD)
    d_all, lvl2v = _make_sc_inter(0, E)(embs, sp_f, lvl2_f, idx)
    prob, ps, ns = _make_math(E, B_POS // BLK)(
        d_all, lvl2v.reshape(ROWS_E, 128))
    loss = -(ps[0, 0] / B_POS) - (ns[0, 0] / B_NEG)
    edge_prob = prob.reshape(E)
    ground_truth = jnp.concatenate(
        [jnp.ones(B_POS, dtype=jnp.float32), jnp.zeros(B_NEG, dtype=jnp.float32)])
    return loss, edge_prob, ground_truth


# speedup vs baseline: 1.0649x; 1.0180x over previous
import functools

import jax
import jax.numpy as jnp
from jax import lax
from jax.experimental import pallas as pl
from jax.experimental.pallas import tpu as pltpu
from jax.experimental.pallas import tpu_sc as plsc

N_NODES = 100000
DIM = 128
B_POS = 16384
B_NEG = 81920
EPS = 1e-7

E = B_POS + B_NEG
TOT = 2 * E
NW = 32
PER_W = E // NW
CH = 128
N_CH = PER_W // CH

LOG2E = 1.4426950408889634
LN2 = 0.6931471805599453
EPS2 = EPS / LN2

N_PAD = 100096
ROWS_TAB = N_PAD // 128


def _softplus2(x):
    return jnp.log2(1.0 + jnp.exp2(x * LOG2E))


def _tab_body(o_ref, sp_ref, lvl2_ref):
    x = o_ref[...]
    sp = LN2 * _softplus2(x)
    sp_ref[...] = sp
    lvl2_ref[...] = jnp.log2(_softplus2(sp) + EPS2)


_tab_kernel = pl.pallas_call(
    _tab_body,
    out_shape=[
        jax.ShapeDtypeStruct((ROWS_TAB, 128), jnp.float32),
        jax.ShapeDtypeStruct((ROWS_TAB, 128), jnp.float32),
    ],
)


@functools.cache
def _make_sc_inter(ebase, ecount):
    mesh = plsc.VectorSubcoreMesh(core_axis_name="c", subcore_axis_name="s")

    per_w = ecount // NW
    n_ch = per_w // CH
    NB = 3
    scratch_types = [
        pltpu.VMEM((per_w,), jnp.int32),
        pltpu.VMEM((per_w,), jnp.int32),
        pltpu.VMEM((per_w,), jnp.float32),
        pltpu.VMEM((per_w,), jnp.float32),
        pltpu.VMEM((per_w,), jnp.float32),
    ]
    scratch_types += [pltpu.VMEM((CH, DIM), jnp.float32)] * (2 * NB)
    scratch_types += [pltpu.SemaphoreType.DMA] * (3 * NB + 1)

    @functools.partial(
        pl.kernel,
        mesh=mesh,
        out_type=(
            jax.ShapeDtypeStruct((ecount, DIM), jnp.float32),
            jax.ShapeDtypeStruct((ecount,), jnp.float32),
        ),
        scratch_types=scratch_types,
    )
    def _sc_inter(embs_hbm, sp_hbm, lvl2_hbm, idx_hbm, d_out, lvl2_out,
                  idxu, idxv, spu, spv, lvl2v, *scr):
        PER_W = per_w
        N_CH = n_ch
        ubufs = scr[:NB]
        vbufs = scr[NB:2 * NB]
        sgu = scr[2 * NB:3 * NB]
        sgv = scr[3 * NB:4 * NB]
        sws = scr[4 * NB:5 * NB]
        so = scr[5 * NB]
        wid = lax.axis_index("s") * 2 + lax.axis_index("c")
        base = wid * PER_W
        pltpu.sync_copy(idx_hbm.at[pl.ds(ebase + base, PER_W)], idxu)
        pltpu.sync_copy(idx_hbm.at[pl.ds(E + ebase + base, PER_W)], idxv)

        def start_g(c, p):
            sl = pl.ds(c * CH, CH)
            pltpu.async_copy(embs_hbm.at[idxu.at[sl]], ubufs[p], sgu[p])
            pltpu.async_copy(embs_hbm.at[idxv.at[sl]], vbufs[p], sgv[p])

        def wait_g(c, p):
            sl = pl.ds(c * CH, CH)
            pltpu.make_async_copy(embs_hbm.at[idxu.at[sl]], ubufs[p],
                                  sgu[p]).wait()
            pltpu.make_async_copy(embs_hbm.at[idxv.at[sl]], vbufs[p],
                                  sgv[p]).wait()

        def out_at(c):
            return d_out.at[pl.ds(base + c * CH, CH)]

        def start_w(c, p):
            pltpu.async_copy(ubufs[p], out_at(c), sws[p])

        def wait_w(c, p):
            pltpu.make_async_copy(ubufs[p], out_at(c), sws[p]).wait()

        start_g(0, 0)
        def sbody(j, carry):
            sl = pl.ds(j * CH, CH)
            pltpu.async_copy(sp_hbm.at[idxu.at[sl]], spu.at[sl], so)
            pltpu.async_copy(sp_hbm.at[idxv.at[sl]], spv.at[sl], so)
            pltpu.async_copy(lvl2_hbm.at[idxv.at[sl]], lvl2v.at[sl], so)
            return carry

        lax.fori_loop(0, N_CH, sbody, 0)
        pltpu.make_async_copy(sp_hbm.at[pl.ds(0, PER_W)], spu, so).wait()
        pltpu.make_async_copy(sp_hbm.at[pl.ds(0, PER_W)], spv, so).wait()
        pltpu.make_async_copy(lvl2_hbm.at[pl.ds(0, PER_W)], lvl2v, so).wait()

        def cbody(co, carry):
            for kb in range(NB):
                c = co * NB + kb
                pn = (kb + 1) % NB

                @pl.when(c + 1 < N_CH)
                def _():
                    @pl.when(c >= 2)
                    def _():
                        wait_w(c - 2, pn)
                    start_g(c + 1, pn)

                wait_g(c, kb)
                ub = ubufs[kb]
                vb = vbufs[kb]

                def gbody(g, gcarry, ub=ub, vb=vb, c=c):
                    suv = spu[pl.ds(c * CH + g * 16, 16)]
                    svv = spv[pl.ds(c * CH + g * 16, 16)]
                    for j in range(16):
                        r = g * 16 + j
                        su = jnp.full((16,), suv[j], jnp.float32)
                        sv = jnp.full((16,), svv[j], jnp.float32)
                        for k in range(DIM // 16):
                            sl = pl.ds(k * 16, 16)
                            uu = ub[r, sl]
                            vv = vb[r, sl]
                            ub[r, sl] = jnp.minimum(uu + su, vv + sv) \
                                - jnp.maximum(uu, vv)
                    return gcarry

                lax.fori_loop(0, CH // 16, gbody, 0)
                start_w(c, kb)
            return carry

        lax.fori_loop(0, N_CH // NB, cbody, 0)
        for c in range(N_CH - NB, N_CH):
            wait_w(c, c % NB)
        pltpu.sync_copy(lvl2v, lvl2_out.at[pl.ds(base, PER_W)])

    return _sc_inter


ROWS_E = E // 128

BLK = 8192
NBLK = E // BLK
NPOS = B_POS // BLK
BRK = BLK // 128


def _math_body(npos, d_ref, lvl2_ref, prob_ref, gt_ref, loss_ref,
               pos_ref, neg_ref):
    i = pl.program_id(0)
    out_rows = []
    for r in range(BRK):
        dt = d_ref[r * 128:(r + 1) * 128, :].T
        t = jnp.exp2(dt * LOG2E)
        s = jnp.log2(jnp.log2(1.0 + t) + EPS2)
        out_rows.append(jnp.sum(s, axis=0, keepdims=True))
    sums = jnp.concatenate(out_rows, axis=0)
    logp = LN2 * (sums - DIM * lvl2_ref[...])
    prob = jax.nn.sigmoid(logp)
    prob_ref[...] = prob
    is_pos = i < npos
    gt_ref[...] = jnp.full((BRK, 128), jnp.where(is_pos, 1.0, 0.0),
                           jnp.float32)
    lp = jnp.sum(jnp.log(prob + EPS))
    ln = jnp.sum(jnp.log(1.0 - prob + EPS))

    @pl.when(i == 0)
    def _():
        pos_ref[0] = 0.0
        neg_ref[0] = 0.0

    pos_ref[0] += jnp.where(is_pos, lp, 0.0)
    neg_ref[0] += jnp.where(is_pos, 0.0, ln)

    @pl.when(i == NBLK - 1)
    def _():
        loss_ref[0] = -(pos_ref[0] / B_POS) - (neg_ref[0] / B_NEG)


_math_kernel = pl.pallas_call(
    functools.partial(_math_body, B_POS // BLK),
    grid=(NBLK,),
    in_specs=[
        pl.BlockSpec((BLK, DIM), lambda i: (i, 0)),
        pl.BlockSpec((BRK, 128), lambda i: (i, 0)),
    ],
    out_specs=[
        pl.BlockSpec((BRK, 128), lambda i: (i, 0)),
        pl.BlockSpec((BRK, 128), lambda i: (i, 0)),
        pl.BlockSpec(memory_space=pltpu.SMEM),
        pl.BlockSpec(memory_space=pltpu.SMEM),
        pl.BlockSpec(memory_space=pltpu.SMEM),
    ],
    out_shape=[
        jax.ShapeDtypeStruct((ROWS_E, 128), jnp.float32),
        jax.ShapeDtypeStruct((ROWS_E, 128), jnp.float32),
        jax.ShapeDtypeStruct((1,), jnp.float32),
        jax.ShapeDtypeStruct((1,), jnp.float32),
        jax.ShapeDtypeStruct((1,), jnp.float32),
    ],
)


def kernel(embs, offset_embs, pos_u, pos_v, neg_u, neg_v):
    idx = jnp.concatenate([pos_u, neg_u, pos_v, neg_v]).astype(jnp.int32)
    off_pad = jnp.pad(offset_embs.reshape(N_NODES),
                      (0, N_PAD - N_NODES)).reshape(ROWS_TAB, 128)
    sp_t, lvl2_t = _tab_kernel(off_pad)
    sp_f = sp_t.reshape(N_PAD)
    lvl2_f = lvl2_t.reshape(N_PAD)
    d_all, lvl2v = _make_sc_inter(0, E)(embs, sp_f, lvl2_f, idx)
    prob, gt, loss, _, _ = _math_kernel(d_all, lvl2v.reshape(ROWS_E, 128))
    return loss[0], prob.reshape(E), gt.reshape(E)
